# baseline (speedup 1.0000x reference)
import jax
import jax.numpy as jnp
from jax import lax
from jax.experimental import pallas as pl
from jax.experimental.pallas import tpu as pltpu, tpu_sc as plsc

_NC = 2
_NS = 16
_NW = _NC * _NS
_LANES = 16

_B = 16384
_L = 200
_N = _B * _L
_PER_W = _N // _NW
_CH = 6400
_NCH = _PER_W // _CH


def _sc_body(xq_hbm, emb_hbm, w_hbm, b_hbm, out_hbm,
             emb_v, w_v, b_v, t_v, p_v, xin0, xin1,
             g00, g01, g02, g03, g04, g10, g11, g12, g13, g14,
             isem0, isem1, osem0, osem1):
    wid = lax.axis_index("s") * _NC + lax.axis_index("c")
    lane = lax.iota(jnp.int32, _LANES)
    zero16 = jnp.zeros((_LANES,), jnp.int32)
    base_w = wid * _PER_W

    pltpu.make_async_copy(
        xq_hbm.at[pl.ds(base_w, _CH)], xin0, isem0).start()
    pltpu.make_async_copy(
        xq_hbm.at[pl.ds(base_w + _CH, _CH)], xin1, isem1).start()

    pltpu.sync_copy(emb_hbm, emb_v)
    pltpu.sync_copy(w_hbm, w_v)
    pltpu.sync_copy(b_hbm, b_v)

    for chunk in range(4):
        n = chunk * _LANES + lane
        v = jnp.minimum(n // 5, 9)
        c = n - 5 * (n // 5)
        acc = jnp.zeros((_LANES,), jnp.float32)
        for d in range(20):
            dvec = zero16 + d
            e = plsc.load_gather(emb_v, [v, dvec])
            w = plsc.load_gather(w_v, [c, dvec])
            acc = acc + e * w
        acc = acc + plsc.load_gather(b_v, [zero16, c])
        t_v[pl.ds(chunk * _LANES, _LANES)] = acc

    def pbuild(k, carry):
        n = k * _LANES + lane
        p = n // 10
        j = n - 10 * p
        hi = jnp.minimum(p // 10, 9)
        lo = p - 10 * (p // 10)
        src = jnp.where(j < 5, hi * 5 + j, lo * 5 + (j - 5))
        val = plsc.load_gather(t_v, [src])
        plsc.store_scatter(p_v, [n], val)
        return carry
    lax.fori_loop(0, 64, pbuild, 0)

    xin = [xin0, xin1]
    stg = [[g00, g01, g02, g03, g04], [g10, g11, g12, g13, g14]]
    isem = [isem0, isem1]
    osem = [osem0, osem1]

    def in_copy(ch, bank):
        return pltpu.make_async_copy(
            xq_hbm.at[pl.ds(base_w + ch * _CH, _CH)], xin[bank], isem[bank])

    def out_copies(ch, bank):
        return [
            pltpu.make_async_copy(
                stg[bank][c],
                out_hbm.at[pl.ds(c * _N + base_w + ch * _CH, _CH)],
                osem[bank])
            for c in range(5)
        ]

    def compute(bank):
        xb = xin[bank]
        sb = stg[bank]

        @plsc.parallel_loop(0, _CH // 32, unroll=4)
        def pair_iter(k):
            ev = xb[pl.ds(k * 32, _LANES)]
            od = xb[pl.ds(k * 32 + _LANES, _LANES)]
            addr = ev * 100 + od * 10
            je = k * 32 + lane
            jo = je + _LANES
            for c in range(5):
                ve = plsc.load_gather(p_v, [addr + c])
                plsc.store_scatter(sb[c], [je], ve)
                vo = plsc.load_gather(p_v, [addr + (5 + c)])
                plsc.store_scatter(sb[c], [jo], vo)

    def pair_body(i, carry):
        ch0 = 2 * i
        in_copy(ch0, 0).wait()

        @pl.when(i > 0)
        def _():
            for cp in out_copies(ch0 - 2, 0):
                cp.wait()
        compute(0)
        for cp in out_copies(ch0, 0):
            cp.start()

        @pl.when(ch0 + 2 < _NCH)
        def _():
            in_copy(ch0 + 2, 0).start()

        in_copy(ch0 + 1, 1).wait()

        @pl.when(i > 0)
        def _():
            for cp in out_copies(ch0 - 1, 1):
                cp.wait()
        compute(1)
        for cp in out_copies(ch0 + 1, 1):
            cp.start()

        @pl.when(ch0 + 3 < _NCH)
        def _():
            in_copy(ch0 + 3, 1).start()
        return carry

    lax.fori_loop(0, _NCH // 2, pair_body, 0)
    for cp in out_copies(_NCH - 2, 0):
        cp.wait()
    for cp in out_copies(_NCH - 1, 1):
        cp.wait()


def kernel(x, emb, W, b):
    B, L = x.shape
    xq = (x.T.reshape(_L // 8, 8, _B // 128, 128)
          .transpose(0, 2, 1, 3).reshape(_N))
    mesh = plsc.VectorSubcoreMesh(core_axis_name="c", subcore_axis_name="s")
    run = pl.kernel(
        _sc_body,
        out_type=jax.ShapeDtypeStruct((5 * _N,), jnp.float32),
        mesh=mesh,
        compiler_params=pltpu.CompilerParams(needs_layout_passes=False),
        scratch_types=[
            pltpu.VMEM((10, 20), jnp.float32),
            pltpu.VMEM((5, 20), jnp.float32),
            pltpu.VMEM((1, 5), jnp.float32),
            pltpu.VMEM((64,), jnp.float32),
            pltpu.VMEM((1024,), jnp.float32),
            pltpu.VMEM((_CH,), jnp.int32),
            pltpu.VMEM((_CH,), jnp.int32),
        ] + [pltpu.VMEM((_CH,), jnp.float32)] * 10 + [
            pltpu.SemaphoreType.DMA,
            pltpu.SemaphoreType.DMA,
            pltpu.SemaphoreType.DMA,
            pltpu.SemaphoreType.DMA,
        ],
    )
    of = run(xq, emb, W, b)
    return (of.reshape(5, _L // 8, _B // 128, 8, 128)
            .transpose(2, 4, 1, 3, 0).reshape(B, L, 5))


# speedup vs baseline: 1.1061x; 1.1061x over previous
import jax
import jax.numpy as jnp
from jax import lax
from jax.experimental import pallas as pl
from jax.experimental.pallas import tpu as pltpu, tpu_sc as plsc

_NC = 2
_NS = 16
_NW = _NC * _NS
_LANES = 16

_B = 16384
_L = 200
_N = _B * _L
_PER_W = _N // _NW
_CH = 6400
_NCH = _PER_W // _CH


def _sc_body(xq_hbm, emb_hbm, w_hbm, b_hbm, out_hbm,
             emb_v, w_v, b_v, t_v, p_v, xin0, xin1,
             g00, g01, g02, g03, g04, g10, g11, g12, g13, g14,
             isem0, isem1, osem0, osem1):
    wid = lax.axis_index("s") * _NC + lax.axis_index("c")
    lane = lax.iota(jnp.int32, _LANES)
    zero16 = jnp.zeros((_LANES,), jnp.int32)

    pltpu.sync_copy(emb_hbm, emb_v)
    pltpu.sync_copy(w_hbm, w_v)
    pltpu.sync_copy(b_hbm, b_v)

    for chunk in range(4):
        n = chunk * _LANES + lane
        v = jnp.minimum(n // 5, 9)
        c = n - 5 * (n // 5)
        acc = jnp.zeros((_LANES,), jnp.float32)
        for d in range(20):
            dvec = zero16 + d
            e = plsc.load_gather(emb_v, [v, dvec])
            w = plsc.load_gather(w_v, [c, dvec])
            acc = acc + e * w
        acc = acc + plsc.load_gather(b_v, [zero16, c])
        t_v[pl.ds(chunk * _LANES, _LANES)] = acc

    def pbuild(k, carry):
        n = k * _LANES + lane
        p = n // 10
        j = n - 10 * p
        hi = jnp.minimum(p // 10, 9)
        lo = p - 10 * (p // 10)
        src = jnp.where(j < 5, hi * 5 + j, lo * 5 + (j - 5))
        val = plsc.load_gather(t_v, [src])
        plsc.store_scatter(p_v, [n], val)
        return carry
    lax.fori_loop(0, 64, pbuild, 0)

    xin = [xin0, xin1]
    stg = [[g00, g01, g02, g03, g04], [g10, g11, g12, g13, g14]]
    isem = [isem0, isem1]
    osem = [osem0, osem1]
    base_w = wid * _PER_W

    def in_copy(ch, bank):
        return pltpu.make_async_copy(
            xq_hbm.at[pl.ds(base_w + ch * _CH, _CH)], xin[bank], isem[bank])

    def out_copies(ch, bank):
        return [
            pltpu.make_async_copy(
                stg[bank][c],
                out_hbm.at[pl.ds(c * _N + base_w + ch * _CH, _CH)],
                osem[bank])
            for c in range(5)
        ]

    def compute(bank):
        xb = xin[bank]
        sb = stg[bank]

        @plsc.parallel_loop(0, _CH // 32, unroll=2)
        def pair_iter(k):
            ev = xb[pl.ds(k * 32, _LANES)]
            od = xb[pl.ds(k * 32 + _LANES, _LANES)]
            addr = ev * 100 + od * 10
            je = k * 32 + lane
            jo = je + _LANES
            for c in range(5):
                ve = plsc.load_gather(p_v, [addr + c])
                plsc.store_scatter(sb[c], [je], ve)
                vo = plsc.load_gather(p_v, [addr + (5 + c)])
                plsc.store_scatter(sb[c], [jo], vo)

    in_copy(0, 0).start()

    def pair_body(i, carry):
        ch0 = 2 * i
        in_copy(ch0 + 1, 1).start()
        in_copy(ch0, 0).wait()

        @pl.when(i > 0)
        def _():
            for cp in out_copies(ch0 - 2, 0):
                cp.wait()
        compute(0)
        for cp in out_copies(ch0, 0):
            cp.start()

        @pl.when(ch0 + 2 < _NCH)
        def _():
            in_copy(ch0 + 2, 0).start()
        in_copy(ch0 + 1, 1).wait()

        @pl.when(i > 0)
        def _():
            for cp in out_copies(ch0 - 1, 1):
                cp.wait()
        compute(1)
        for cp in out_copies(ch0 + 1, 1):
            cp.start()
        return carry

    lax.fori_loop(0, _NCH // 2, pair_body, 0)
    for cp in out_copies(_NCH - 2, 0):
        cp.wait()
    for cp in out_copies(_NCH - 1, 1):
        cp.wait()


def kernel(x, emb, W, b):
    B, L = x.shape
    xq = (x.T.reshape(_L // 8, 8, _B // 128, 128)
          .transpose(0, 2, 1, 3).reshape(_N))
    mesh = plsc.VectorSubcoreMesh(core_axis_name="c", subcore_axis_name="s")
    run = pl.kernel(
        _sc_body,
        out_type=jax.ShapeDtypeStruct((5 * _N,), jnp.float32),
        mesh=mesh,
        compiler_params=pltpu.CompilerParams(needs_layout_passes=False),
        scratch_types=[
            pltpu.VMEM((10, 20), jnp.float32),
            pltpu.VMEM((5, 20), jnp.float32),
            pltpu.VMEM((1, 5), jnp.float32),
            pltpu.VMEM((64,), jnp.float32),
            pltpu.VMEM((1024,), jnp.float32),
            pltpu.VMEM((_CH,), jnp.int32),
            pltpu.VMEM((_CH,), jnp.int32),
        ] + [pltpu.VMEM((_CH,), jnp.float32)] * 10 + [
            pltpu.SemaphoreType.DMA,
            pltpu.SemaphoreType.DMA,
            pltpu.SemaphoreType.DMA,
            pltpu.SemaphoreType.DMA,
        ],
    )
    of = run(xq, emb, W, b)
    return (of.reshape(5, _L // 8, _B // 128, 8, 128)
            .transpose(2, 4, 1, 3, 0).reshape(B, L, 5))


# speedup vs baseline: 1.1235x; 1.0157x over previous
import jax
import jax.numpy as jnp
from jax import lax
from jax.experimental import pallas as pl
from jax.experimental.pallas import tpu as pltpu, tpu_sc as plsc

_NC = 2
_NS = 16
_NW = _NC * _NS
_LANES = 16

_B = 16384
_L = 200
_N = _B * _L
_PER_W = _N // _NW
_CH = 6400
_NCH = _PER_W // _CH


def _sc_body(xq_hbm, emb_hbm, w_hbm, b_hbm, out_hbm,
             emb_v, w_v, b_v, t_v, p_v, xin0, xin1,
             g00, g01, g02, g03, g04, g10, g11, g12, g13, g14,
             isem0, isem1, osem0, osem1):
    wid = lax.axis_index("s") * _NC + lax.axis_index("c")
    lane = lax.iota(jnp.int32, _LANES)
    zero16 = jnp.zeros((_LANES,), jnp.int32)

    pltpu.sync_copy(emb_hbm, emb_v)
    pltpu.sync_copy(w_hbm, w_v)
    pltpu.sync_copy(b_hbm, b_v)

    for chunk in range(4):
        n = chunk * _LANES + lane
        v = jnp.minimum(n // 5, 9)
        c = n - 5 * (n // 5)
        acc = jnp.zeros((_LANES,), jnp.float32)
        for d in range(20):
            dvec = zero16 + d
            e = plsc.load_gather(emb_v, [v, dvec])
            w = plsc.load_gather(w_v, [c, dvec])
            acc = acc + e * w
        acc = acc + plsc.load_gather(b_v, [zero16, c])
        t_v[pl.ds(chunk * _LANES, _LANES)] = acc

    def pbuild(k, carry):
        n = k * _LANES + lane
        p = n // 10
        j = n - 10 * p
        hi = jnp.minimum(p // 10, 9)
        lo = p - 10 * (p // 10)
        src = jnp.where(j < 5, hi * 5 + j, lo * 5 + (j - 5))
        val = plsc.load_gather(t_v, [src])
        plsc.store_scatter(p_v, [n], val)
        return carry
    lax.fori_loop(0, 64, pbuild, 0)

    xin = [xin0, xin1]
    stg = [[g00, g01, g02, g03, g04], [g10, g11, g12, g13, g14]]
    isem = [isem0, isem1]
    osem = [osem0, osem1]
    base_w = wid * _PER_W

    def in_copy(ch, bank):
        return pltpu.make_async_copy(
            xq_hbm.at[pl.ds(base_w + ch * _CH, _CH)], xin[bank], isem[bank])

    def out_copies(ch, bank):
        return [
            pltpu.make_async_copy(
                stg[bank][c],
                out_hbm.at[pl.ds(c * _N + base_w + ch * _CH, _CH)],
                osem[bank])
            for c in range(5)
        ]

    def compute(bank):
        xb = xin[bank]
        sb = stg[bank]

        @plsc.parallel_loop(0, _CH // 32, unroll=1)
        def pair_iter(k):
            ev = xb[pl.ds(k * 32, _LANES)]
            od = xb[pl.ds(k * 32 + _LANES, _LANES)]
            addr = ev * 100 + od * 10
            je = k * 32 + lane
            jo = je + _LANES
            for c in range(5):
                ve = plsc.load_gather(p_v, [addr + c])
                plsc.store_scatter(sb[c], [je], ve)
                vo = plsc.load_gather(p_v, [addr + (5 + c)])
                plsc.store_scatter(sb[c], [jo], vo)

    in_copy(0, 0).start()

    def pair_body(i, carry):
        ch0 = 2 * i
        in_copy(ch0 + 1, 1).start()
        in_copy(ch0, 0).wait()

        @pl.when(i > 0)
        def _():
            for cp in out_copies(ch0 - 2, 0):
                cp.wait()
        compute(0)
        for cp in out_copies(ch0, 0):
            cp.start()

        @pl.when(ch0 + 2 < _NCH)
        def _():
            in_copy(ch0 + 2, 0).start()
        in_copy(ch0 + 1, 1).wait()

        @pl.when(i > 0)
        def _():
            for cp in out_copies(ch0 - 1, 1):
                cp.wait()
        compute(1)
        for cp in out_copies(ch0 + 1, 1):
            cp.start()
        return carry

    lax.fori_loop(0, _NCH // 2, pair_body, 0)
    for cp in out_copies(_NCH - 2, 0):
        cp.wait()
    for cp in out_copies(_NCH - 1, 1):
        cp.wait()


def kernel(x, emb, W, b):
    B, L = x.shape
    xq = (x.T.reshape(_L // 8, 8, _B // 128, 128)
          .transpose(0, 2, 1, 3).reshape(_N))
    mesh = plsc.VectorSubcoreMesh(core_axis_name="c", subcore_axis_name="s")
    run = pl.kernel(
        _sc_body,
        out_type=jax.ShapeDtypeStruct((5 * _N,), jnp.float32),
        mesh=mesh,
        compiler_params=pltpu.CompilerParams(needs_layout_passes=False),
        scratch_types=[
            pltpu.VMEM((10, 20), jnp.float32),
            pltpu.VMEM((5, 20), jnp.float32),
            pltpu.VMEM((1, 5), jnp.float32),
            pltpu.VMEM((64,), jnp.float32),
            pltpu.VMEM((1024,), jnp.float32),
            pltpu.VMEM((_CH,), jnp.int32),
            pltpu.VMEM((_CH,), jnp.int32),
        ] + [pltpu.VMEM((_CH,), jnp.float32)] * 10 + [
            pltpu.SemaphoreType.DMA,
            pltpu.SemaphoreType.DMA,
            pltpu.SemaphoreType.DMA,
            pltpu.SemaphoreType.DMA,
        ],
    )
    of = run(xq, emb, W, b)
    return (of.reshape(5, _L // 8, _B // 128, 8, 128)
            .transpose(2, 4, 1, 3, 0).reshape(B, L, 5))
